# baseline (speedup 1.0000x reference)
import functools

import jax
import jax.numpy as jnp
from jax import lax
from jax.experimental import pallas as pl
from jax.experimental.pallas import tpu as pltpu
from jax.experimental.pallas import tpu_sc as plsc

_NC = 2
_NS = 16
_K = 104
_EPT = 10192

_N = 10000
_D = 128
_NP = 10240
_DUMMY = _N


def _degree_body(dst_hbm, out_hbm, deg_sh, zbuf, ones_v, idx0, idx1, idx2,
                 is0, is1, is2, ss0, ss1, ss2):
    c = lax.axis_index("c")
    s = lax.axis_index("s")
    idxs = (idx0, idx1, idx2)
    isems = (is0, is1, is2)
    ssems = (ss0, ss1, ss2)

    def _fill(i, _):
        zbuf[pl.ds(i * 16, 16)] = jnp.zeros((16,), jnp.float32)
        return 0

    lax.fori_loop(0, 40, _fill, 0)

    def _fill1(i, _):
        ones_v[pl.ds(i * 16, 16)] = jnp.ones((16,), jnp.float32)
        return 0

    lax.fori_loop(0, _K // 16, _fill1, 0)

    pltpu.sync_copy(zbuf, deg_sh.at[pl.ds(s * 640, 640)])
    plsc.subcore_barrier()

    base = (c * _NS + s) * _EPT

    def _start(k, b):
        pltpu.async_copy(dst_hbm.at[pl.ds(base + k * _K, _K)], idxs[b],
                         isems[b])

    def _wait(b):
        pltpu.make_async_copy(dst_hbm.at[pl.ds(base, _K)], idxs[b],
                              isems[b]).wait()

    def _scat_start(b):
        pltpu.async_copy(ones_v, deg_sh.at[idxs[b]], ssems[b], add=True)

    def _scat_wait(b):
        pltpu.make_async_copy(ones_v, deg_sh.at[idxs[b]], ssems[b]).wait()

    _start(0, 0)
    _start(1, 1)
    _wait(0)
    _scat_start(0)
    _start(2, 2)

    def _chunk(k, b):
        _wait(b)
        _scat_start(b)
        b2 = (b + 2) % 3
        _scat_wait(b2)
        _start(k + 2, b2)

    def _triple(i, _):
        for r in range(3):
            _chunk(1 + 3 * i + r, (1 + r) % 3)
        return 0

    lax.fori_loop(0, (_CPT - 5) // 3, _triple, 0)
    _chunk(_CPT - 4, 1)
    _chunk(_CPT - 3, 2)
    _wait(0)
    _scat_start(0)
    _wait(1)
    _scat_start(1)
    _scat_wait(2)
    _scat_wait(0)
    _scat_wait(1)

    plsc.subcore_barrier()
    pltpu.sync_copy(deg_sh.at[pl.ds(s * 640, 640)],
                    out_hbm.at[pl.ds(c * _NP + s * 640, 640)])


def _sc_degree(dst):
    kern = pl.kernel(
        _degree_body,
        out_type=jax.ShapeDtypeStruct((_NC * _NP,), jnp.float32),
        mesh=plsc.VectorSubcoreMesh(core_axis_name="c", subcore_axis_name="s"),
        scratch_types=[
            pltpu.VMEM_SHARED((_NP,), jnp.float32),
            pltpu.VMEM((640,), jnp.float32),
            pltpu.VMEM((_K,), jnp.float32),
            pltpu.VMEM((_K,), jnp.int32),
            pltpu.VMEM((_K,), jnp.int32),
            pltpu.VMEM((_K,), jnp.int32),
            pltpu.SemaphoreType.DMA,
            pltpu.SemaphoreType.DMA,
            pltpu.SemaphoreType.DMA,
            pltpu.SemaphoreType.DMA,
            pltpu.SemaphoreType.DMA,
            pltpu.SemaphoreType.DMA,
        ],
    )
    return kern(dst)


_CPT = _EPT // _K


def _agg_body(g_hbm, src4_hbm, dst_hbm, out_hbm, agg_sh, src_big, rows0,
              rows1, rows2, dst0, dst1, dst2, sem_e, gs0, gs1, gs2, ds0,
              ds1, ds2, ss0, ss1, ss2):
    c = lax.axis_index("c")
    s = lax.axis_index("s")
    rows = (rows0, rows1, rows2)
    gsems = (gs0, gs1, gs2)
    dsts = (dst0, dst1, dst2)
    dsems = (ds0, ds1, ds2)
    ssems = (ss0, ss1, ss2)

    ebase = (c * _NS + s) * _EPT
    pltpu.async_copy(src4_hbm.at[pl.ds(ebase, _EPT)], src_big, sem_e)

    def _fill(i, _):
        for j in range(_D // 16):
            rows0[i, pl.ds(j * 16, 16)] = jnp.zeros((16,), jnp.float32)
        return 0

    lax.fori_loop(0, _K, _fill, 0)

    @pl.when(s < 10)
    def _zero():
        for t in range(12):
            pltpu.sync_copy(rows0,
                            agg_sh.at[pl.ds(s * 1000 + t * _K, _K)])
        pltpu.sync_copy(rows0.at[pl.ds(0, 40)],
                        agg_sh.at[pl.ds(s * 1000 + 960, 40)])

    pltpu.make_async_copy(src4_hbm.at[pl.ds(ebase, _EPT)], src_big,
                          sem_e).wait()
    plsc.subcore_barrier()

    def _start(k, b):
        pltpu.async_copy(g_hbm.at[src_big.at[pl.ds(k * _K, _K)]], rows[b],
                         gsems[b])
        pltpu.async_copy(dst_hbm.at[pl.ds(ebase + k * _K, _K)], dsts[b],
                         dsems[b])

    def _wait(b):
        pltpu.make_async_copy(g_hbm.at[src_big.at[pl.ds(0, _K)]], rows[b],
                              gsems[b]).wait()
        pltpu.make_async_copy(dst_hbm.at[pl.ds(ebase, _K)], dsts[b],
                              dsems[b]).wait()

    def _scat_start(b):
        pltpu.async_copy(rows[b], agg_sh.at[dsts[b]], ssems[b], add=True)

    def _scat_wait(b):
        pltpu.make_async_copy(rows[b], agg_sh.at[dsts[b]], ssems[b]).wait()

    _start(0, 0)
    _start(1, 1)
    _wait(0)
    _scat_start(0)
    _start(2, 2)

    def _chunk(k, b):
        _wait(b)
        _scat_start(b)
        b2 = (b + 2) % 3
        _scat_wait(b2)
        _start(k + 2, b2)

    def _triple(i, _):
        for r in range(3):
            _chunk(1 + 3 * i + r, (1 + r) % 3)
        return 0

    lax.fori_loop(0, (_CPT - 5) // 3, _triple, 0)
    _chunk(_CPT - 4, 1)
    _chunk(_CPT - 3, 2)
    _wait(0)
    _scat_start(0)
    _wait(1)
    _scat_start(1)
    _scat_wait(2)
    _scat_wait(0)
    _scat_wait(1)

    plsc.subcore_barrier()

    @pl.when(s < 10)
    def _out():
        pltpu.sync_copy(agg_sh.at[pl.ds(s * 1000, 1000)],
                        out_hbm.at[c, pl.ds(s * 1000, 1000)])


def _sc_aggregate(g, src4, dst):
    kern = pl.kernel(
        _agg_body,
        out_type=jax.ShapeDtypeStruct((_NC, _N, _D), jnp.float32),
        mesh=plsc.VectorSubcoreMesh(core_axis_name="c", subcore_axis_name="s"),
        scratch_types=[
            pltpu.VMEM_SHARED((_N + 8, _D), jnp.float32),
            pltpu.VMEM((_EPT,), jnp.int32),
            pltpu.VMEM((_K, _D), jnp.float32),
            pltpu.VMEM((_K, _D), jnp.float32),
            pltpu.VMEM((_K, _D), jnp.float32),
            pltpu.VMEM((_K,), jnp.int32),
            pltpu.VMEM((_K,), jnp.int32),
            pltpu.VMEM((_K,), jnp.int32),
            pltpu.SemaphoreType.DMA,
            pltpu.SemaphoreType.DMA,
            pltpu.SemaphoreType.DMA,
            pltpu.SemaphoreType.DMA,
            pltpu.SemaphoreType.DMA,
            pltpu.SemaphoreType.DMA,
            pltpu.SemaphoreType.DMA,
            pltpu.SemaphoreType.DMA,
            pltpu.SemaphoreType.DMA,
            pltpu.SemaphoreType.DMA,
        ],
    )
    return kern(g, src4, dst)


_ROWS = 1000


def _tc1_body(x_ref, w_ref, d_ref, g_ref, dinv_ref):
    dinv = lax.rsqrt(d_ref[...])
    h = jnp.dot(x_ref[...], w_ref[...], preferred_element_type=jnp.float32)
    g_ref[...] = dinv * h
    dinv_ref[...] = dinv


def _tc1(x, w1, degsum):
    return pl.pallas_call(
        _tc1_body,
        grid=(_N // _ROWS,),
        in_specs=[
            pl.BlockSpec((_ROWS, _D), lambda i: (i, 0)),
            pl.BlockSpec((_D, _D), lambda i: (0, 0)),
            pl.BlockSpec((_ROWS, 1), lambda i: (i, 0)),
        ],
        out_specs=[
            pl.BlockSpec((_ROWS, _D), lambda i: (i, 0)),
            pl.BlockSpec((_ROWS, 1), lambda i: (i, 0)),
        ],
        out_shape=[
            jax.ShapeDtypeStruct((_N, _D), jnp.float32),
            jax.ShapeDtypeStruct((_N, 1), jnp.float32),
        ],
    )(x, w1, degsum)


def _tc2_body(g_ref, a_ref, dinv_ref, b_ref, w_ref, g2_ref):
    dinv = dinv_ref[...]
    z = dinv * (g_ref[...] + a_ref[0] + a_ref[1]) + b_ref[...]
    z = jnp.maximum(z, 0.0)
    h2 = jnp.dot(z, w_ref[...], preferred_element_type=jnp.float32)
    g2_ref[...] = dinv * h2


def _tc2(g1, agg, dinv, b1, w2):
    return pl.pallas_call(
        _tc2_body,
        grid=(_N // _ROWS,),
        in_specs=[
            pl.BlockSpec((_ROWS, _D), lambda i: (i, 0)),
            pl.BlockSpec((_NC, _ROWS, _D), lambda i: (0, i, 0)),
            pl.BlockSpec((_ROWS, 1), lambda i: (i, 0)),
            pl.BlockSpec((1, _D), lambda i: (0, 0)),
            pl.BlockSpec((_D, _D), lambda i: (0, 0)),
        ],
        out_specs=pl.BlockSpec((_ROWS, _D), lambda i: (i, 0)),
        out_shape=jax.ShapeDtypeStruct((_N, _D), jnp.float32),
    )(g1, agg, dinv, b1, w2)


def _tc3_body(g_ref, a_ref, dinv_ref, b_ref, o_ref):
    o_ref[...] = (dinv_ref[...] * (g_ref[...] + a_ref[0] + a_ref[1])
                  + b_ref[...])


def _tc3(g2, agg, dinv, b2):
    return pl.pallas_call(
        _tc3_body,
        grid=(_N // _ROWS,),
        in_specs=[
            pl.BlockSpec((_ROWS, _D), lambda i: (i, 0)),
            pl.BlockSpec((_NC, _ROWS, _D), lambda i: (0, i, 0)),
            pl.BlockSpec((_ROWS, 1), lambda i: (i, 0)),
            pl.BlockSpec((1, _D), lambda i: (0, 0)),
        ],
        out_specs=pl.BlockSpec((_ROWS, _D), lambda i: (i, 0)),
        out_shape=jax.ShapeDtypeStruct((_N, _D), jnp.float32),
    )(g2, agg, dinv, b2)


def kernel(x, edge_index, W1, b1, W2, b2):
    ei = edge_index.astype(jnp.int32)
    n_e = ei.shape[1]
    n_t = _NC * _NS
    per_t = n_e // n_t
    srcr = ei[0].reshape(n_t, per_t)
    dstr = ei[1].reshape(n_t, per_t)
    pad = _EPT - per_t
    src4 = jnp.concatenate(
        [srcr, jnp.zeros((n_t, pad), jnp.int32)], axis=1).reshape(-1)
    dst = jnp.concatenate(
        [dstr, jnp.full((n_t, pad), _DUMMY, jnp.int32)], axis=1).reshape(-1)

    deg_p = _sc_degree(dst)
    degsum = (deg_p[:_N] + deg_p[_NP:_NP + _N] + 1.0).reshape(_N, 1)

    g1, dinv = _tc1(x, W1, degsum)
    agg1 = _sc_aggregate(g1, src4, dst)
    g2 = _tc2(g1, agg1, dinv, b1.reshape(1, _D), W2)
    agg2 = _sc_aggregate(g2, src4, dst)
    return _tc3(g2, agg2, dinv, b2.reshape(1, _D))


# speedup vs baseline: 17.9939x; 17.9939x over previous
import functools

import jax
import jax.numpy as jnp
from jax import lax
from jax.experimental import pallas as pl
from jax.experimental.pallas import tpu as pltpu
from jax.experimental.pallas import tpu_sc as plsc

_NC = 2
_NS = 16
_K = 80

_N = 10000
_D = 128
_NP = 10240


def _degree_body(dst_hbm, out_hbm, deg_sh, zbuf, ones_v, idx0, idx1, idx2,
                 is0, is1, is2, ss0, ss1, ss2):
    c = lax.axis_index("c")
    s = lax.axis_index("s")
    idxs = (idx0, idx1, idx2)
    isems = (is0, is1, is2)
    ssems = (ss0, ss1, ss2)

    def _fill(i, _):
        zbuf[pl.ds(i * 16, 16)] = jnp.zeros((16,), jnp.float32)
        return 0

    lax.fori_loop(0, 40, _fill, 0)

    def _fill1(i, _):
        ones_v[pl.ds(i * 16, 16)] = jnp.ones((16,), jnp.float32)
        return 0

    lax.fori_loop(0, _K // 16, _fill1, 0)

    pltpu.sync_copy(zbuf, deg_sh.at[pl.ds(s * 640, 640)])
    plsc.subcore_barrier()

    n_e = dst_hbm.shape[0] // (_NC * _NS)
    base = (c * _NS + s) * n_e

    def _start(k, b):
        pltpu.async_copy(dst_hbm.at[pl.ds(base + k * _K, _K)], idxs[b],
                         isems[b])

    def _wait(b):
        pltpu.make_async_copy(dst_hbm.at[pl.ds(base, _K)], idxs[b],
                              isems[b]).wait()

    def _scat_start(b):
        pltpu.async_copy(ones_v, deg_sh.at[idxs[b]], ssems[b], add=True)

    def _scat_wait(b):
        pltpu.make_async_copy(ones_v, deg_sh.at[idxs[b]], ssems[b]).wait()

    _start(0, 0)
    _start(1, 1)
    _wait(0)
    _scat_start(0)
    _start(2, 2)

    def _chunk(k, b):
        _wait(b)
        _scat_start(b)
        b2 = (b + 2) % 3
        _scat_wait(b2)
        _start(k + 2, b2)

    def _triple(i, _):
        for r in range(3):
            _chunk(1 + 3 * i + r, (1 + r) % 3)
        return 0

    lax.fori_loop(0, (_CPT - 5) // 3, _triple, 0)
    _chunk(_CPT - 4, 1)
    _chunk(_CPT - 3, 2)
    _wait(0)
    _scat_start(0)
    _wait(1)
    _scat_start(1)
    _scat_wait(2)
    _scat_wait(0)
    _scat_wait(1)

    plsc.subcore_barrier()
    pltpu.sync_copy(deg_sh.at[pl.ds(s * 640, 640)],
                    out_hbm.at[pl.ds(c * _NP + s * 640, 640)])


def _sc_degree(dst):
    kern = pl.kernel(
        _degree_body,
        out_type=jax.ShapeDtypeStruct((_NC * _NP,), jnp.float32),
        mesh=plsc.VectorSubcoreMesh(core_axis_name="c", subcore_axis_name="s"),
        scratch_types=[
            pltpu.VMEM_SHARED((_NP,), jnp.float32),
            pltpu.VMEM((640,), jnp.float32),
            pltpu.VMEM((_K,), jnp.float32),
            pltpu.VMEM((_K,), jnp.int32),
            pltpu.VMEM((_K,), jnp.int32),
            pltpu.VMEM((_K,), jnp.int32),
            pltpu.SemaphoreType.DMA,
            pltpu.SemaphoreType.DMA,
            pltpu.SemaphoreType.DMA,
            pltpu.SemaphoreType.DMA,
            pltpu.SemaphoreType.DMA,
            pltpu.SemaphoreType.DMA,
        ],
    )
    return kern(dst)


_CPT = 320000 // (_NC * _NS) // _K


def _agg_body(g_hbm, src4_hbm, dst_hbm, out_hbm, agg_sh, src_big, rows0,
              rows1, rows2, dst0, dst1, dst2, sem_e, gs0, gs1, gs2, ds0,
              ds1, ds2, ss0, ss1, ss2):
    c = lax.axis_index("c")
    s = lax.axis_index("s")
    rows = (rows0, rows1, rows2)
    gsems = (gs0, gs1, gs2)
    dsts = (dst0, dst1, dst2)
    dsems = (ds0, ds1, ds2)
    ssems = (ss0, ss1, ss2)

    ebase = (c * _NS + s) * _CPT * _K
    pltpu.async_copy(src4_hbm.at[pl.ds(ebase, _CPT * _K)], src_big, sem_e)

    def _fill(i, _):
        for j in range(_D // 16):
            rows0[i, pl.ds(j * 16, 16)] = jnp.zeros((16,), jnp.float32)
        return 0

    lax.fori_loop(0, _K, _fill, 0)

    @pl.when(s < 10)
    def _zero():
        for t in range(12):
            pltpu.sync_copy(rows0,
                            agg_sh.at[pl.ds(s * 1000 + t * _K, _K)])
        pltpu.sync_copy(rows0.at[pl.ds(0, 40)],
                        agg_sh.at[pl.ds(s * 1000 + 960, 40)])

    pltpu.make_async_copy(src4_hbm.at[pl.ds(ebase, _CPT * _K)], src_big,
                          sem_e).wait()
    plsc.subcore_barrier()

    def _start(k, b):
        pltpu.async_copy(g_hbm.at[src_big.at[pl.ds(k * _K, _K)]], rows[b],
                         gsems[b])
        pltpu.async_copy(dst_hbm.at[pl.ds(ebase + k * _K, _K)], dsts[b],
                         dsems[b])

    def _wait(b):
        pltpu.make_async_copy(g_hbm.at[src_big.at[pl.ds(0, _K)]], rows[b],
                              gsems[b]).wait()
        pltpu.make_async_copy(dst_hbm.at[pl.ds(ebase, _K)], dsts[b],
                              dsems[b]).wait()

    def _scat_start(b):
        pltpu.async_copy(rows[b], agg_sh.at[dsts[b]], ssems[b], add=True)

    def _scat_wait(b):
        pltpu.make_async_copy(rows[b], agg_sh.at[dsts[b]], ssems[b]).wait()

    _start(0, 0)
    _start(1, 1)
    _wait(0)
    _scat_start(0)
    _start(2, 2)

    def _chunk(k, b):
        _wait(b)
        _scat_start(b)
        b2 = (b + 2) % 3
        _scat_wait(b2)
        _start(k + 2, b2)

    def _triple(i, _):
        for r in range(3):
            _chunk(1 + 3 * i + r, (1 + r) % 3)
        return 0

    lax.fori_loop(0, (_CPT - 5) // 3, _triple, 0)
    _chunk(_CPT - 4, 1)
    _chunk(_CPT - 3, 2)
    _wait(0)
    _scat_start(0)
    _wait(1)
    _scat_start(1)
    _scat_wait(2)
    _scat_wait(0)
    _scat_wait(1)

    plsc.subcore_barrier()

    @pl.when(s < 10)
    def _out():
        pltpu.sync_copy(agg_sh.at[pl.ds(s * 1000, 1000)],
                        out_hbm.at[c, pl.ds(s * 1000, 1000)])


def _sc_aggregate(g, src4, dst):
    kern = pl.kernel(
        _agg_body,
        out_type=jax.ShapeDtypeStruct((_NC, _N, _D), jnp.float32),
        mesh=plsc.VectorSubcoreMesh(core_axis_name="c", subcore_axis_name="s"),
        scratch_types=[
            pltpu.VMEM_SHARED((_N, _D), jnp.float32),
            pltpu.VMEM((_CPT * _K,), jnp.int32),
            pltpu.VMEM((_K, _D), jnp.float32),
            pltpu.VMEM((_K, _D), jnp.float32),
            pltpu.VMEM((_K, _D), jnp.float32),
            pltpu.VMEM((_K,), jnp.int32),
            pltpu.VMEM((_K,), jnp.int32),
            pltpu.VMEM((_K,), jnp.int32),
            pltpu.SemaphoreType.DMA,
            pltpu.SemaphoreType.DMA,
            pltpu.SemaphoreType.DMA,
            pltpu.SemaphoreType.DMA,
            pltpu.SemaphoreType.DMA,
            pltpu.SemaphoreType.DMA,
            pltpu.SemaphoreType.DMA,
            pltpu.SemaphoreType.DMA,
            pltpu.SemaphoreType.DMA,
            pltpu.SemaphoreType.DMA,
        ],
    )
    return kern(g, src4, dst)


_ROWS = 1000


def _tc1_body(x_ref, w_ref, d_ref, g_ref, dinv_ref):
    dinv = lax.rsqrt(d_ref[...])
    h = jnp.dot(x_ref[...], w_ref[...], preferred_element_type=jnp.float32)
    g_ref[...] = dinv * h
    dinv_ref[...] = dinv


def _tc1(x, w1, degsum):
    return pl.pallas_call(
        _tc1_body,
        grid=(_N // _ROWS,),
        in_specs=[
            pl.BlockSpec((_ROWS, _D), lambda i: (i, 0)),
            pl.BlockSpec((_D, _D), lambda i: (0, 0)),
            pl.BlockSpec((_ROWS, 1), lambda i: (i, 0)),
        ],
        out_specs=[
            pl.BlockSpec((_ROWS, _D), lambda i: (i, 0)),
            pl.BlockSpec((_ROWS, 1), lambda i: (i, 0)),
        ],
        out_shape=[
            jax.ShapeDtypeStruct((_N, _D), jnp.float32),
            jax.ShapeDtypeStruct((_N, 1), jnp.float32),
        ],
    )(x, w1, degsum)


def _tc2_body(g_ref, a_ref, dinv_ref, b_ref, w_ref, g2_ref):
    dinv = dinv_ref[...]
    z = dinv * (g_ref[...] + a_ref[0] + a_ref[1]) + b_ref[...]
    z = jnp.maximum(z, 0.0)
    h2 = jnp.dot(z, w_ref[...], preferred_element_type=jnp.float32)
    g2_ref[...] = dinv * h2


def _tc2(g1, agg, dinv, b1, w2):
    return pl.pallas_call(
        _tc2_body,
        grid=(_N // _ROWS,),
        in_specs=[
            pl.BlockSpec((_ROWS, _D), lambda i: (i, 0)),
            pl.BlockSpec((_NC, _ROWS, _D), lambda i: (0, i, 0)),
            pl.BlockSpec((_ROWS, 1), lambda i: (i, 0)),
            pl.BlockSpec((1, _D), lambda i: (0, 0)),
            pl.BlockSpec((_D, _D), lambda i: (0, 0)),
        ],
        out_specs=pl.BlockSpec((_ROWS, _D), lambda i: (i, 0)),
        out_shape=jax.ShapeDtypeStruct((_N, _D), jnp.float32),
    )(g1, agg, dinv, b1, w2)


def _tc3_body(g_ref, a_ref, dinv_ref, b_ref, o_ref):
    o_ref[...] = (dinv_ref[...] * (g_ref[...] + a_ref[0] + a_ref[1])
                  + b_ref[...])


def _tc3(g2, agg, dinv, b2):
    return pl.pallas_call(
        _tc3_body,
        grid=(_N // _ROWS,),
        in_specs=[
            pl.BlockSpec((_ROWS, _D), lambda i: (i, 0)),
            pl.BlockSpec((_NC, _ROWS, _D), lambda i: (0, i, 0)),
            pl.BlockSpec((_ROWS, 1), lambda i: (i, 0)),
            pl.BlockSpec((1, _D), lambda i: (0, 0)),
        ],
        out_specs=pl.BlockSpec((_ROWS, _D), lambda i: (i, 0)),
        out_shape=jax.ShapeDtypeStruct((_N, _D), jnp.float32),
    )(g2, agg, dinv, b2)


def kernel(x, edge_index, W1, b1, W2, b2):
    ei = edge_index.astype(jnp.int32)
    dst = ei[1]
    n_e = ei.shape[1]
    src4 = ei[0]

    deg_p = _sc_degree(dst)
    degsum = (deg_p[:_N] + deg_p[_NP:_NP + _N] + 1.0).reshape(_N, 1)

    g1, dinv = _tc1(x, W1, degsum)
    agg1 = _sc_aggregate(g1, src4, dst)
    g2 = _tc2(g1, agg1, dinv, b1.reshape(1, _D), W2)
    agg2 = _sc_aggregate(g2, src4, dst)
    return _tc3(g2, agg2, dinv, b2.reshape(1, _D))
